# Initial kernel scaffold; baseline (speedup 1.0000x reference)
import jax
import jax.numpy as jnp
from jax.experimental import pallas as pl


def kernel(boxes, scores):
    raise NotImplementedError("write your pallas kernel here")


# speedup vs baseline: 90.2313x; 90.2313x over previous
import functools

import jax
import jax.numpy as jnp
from jax.experimental import pallas as pl
from jax.experimental.pallas import tpu as pltpu

N = 5000
NUM_CLASSES = 20
NP = 5120
R = 40
C = 128
THRESH = 0.5


def _cls_kernel(st_ref, cls_ref, ssel_ref):
    s = st_ref[...]
    m = jnp.max(s, axis=0, keepdims=True)
    row = jax.lax.broadcasted_iota(jnp.int32, s.shape, 0)
    idx = jnp.min(jnp.where(s == m, row, NUM_CLASSES), axis=0, keepdims=True)
    cls_ref[...] = idx
    ssel_ref[...] = m


def _nms_kernel(x1r, y1r, x2r, y2r, cr,
                x1c, y1c, x2c, y2c, ccol,
                keep_ref, supp_ref):
    x1 = x1r[...]
    y1 = y1r[...]
    x2 = x2r[...]
    y2 = y2r[...]
    c = cr[...]
    areas = (x2 - x1) * (y2 - y1)
    pos = (jax.lax.broadcasted_iota(jnp.int32, (R, C), 0) * C
           + jax.lax.broadcasted_iota(jnp.int32, (R, C), 1))
    supp_ref[...] = jnp.zeros((R, C), jnp.float32)

    def body(t, carry):
        x1i = x1c[pl.ds(t, 1), :]
        y1i = y1c[pl.ds(t, 1), :]
        x2i = x2c[pl.ds(t, 1), :]
        y2i = y2c[pl.ds(t, 1), :]
        ci = ccol[pl.ds(t, 1), :]
        ai = (x2i - x1i) * (y2i - y1i)
        supp = supp_ref[...]
        s_t = jnp.max(jnp.where(pos == t, supp, 0.0))
        xx1 = jnp.maximum(x1, x1i)
        yy1 = jnp.maximum(y1, y1i)
        xx2 = jnp.minimum(x2, x2i)
        yy2 = jnp.minimum(y2, y2i)
        w = jnp.maximum(1e-10, xx2 - xx1)
        h = jnp.maximum(1e-10, yy2 - yy1)
        inter = w * h
        over = inter / ((ai + areas) - inter)
        newsup = (over > THRESH) & (c == ci) & (pos > t)
        supp_ref[...] = jnp.maximum(supp, newsup.astype(jnp.float32) * (1.0 - s_t))
        return carry

    jax.lax.fori_loop(0, N, body, 0, unroll=False)
    valid = pos < N
    keep_ref[...] = jnp.where(valid, 1.0 - supp_ref[...], 0.0)


@jax.jit
def kernel(boxes, scores):
    st = jnp.zeros((NUM_CLASSES, NP), jnp.float32)
    st = st.at[:, :N].set(scores.T)
    cls_p, ssel_p = pl.pallas_call(
        _cls_kernel,
        out_shape=[
            jax.ShapeDtypeStruct((1, NP), jnp.int32),
            jax.ShapeDtypeStruct((1, NP), jnp.float32),
        ],
    )(st)
    cls_inds = cls_p[0, :N]
    ssel = ssel_p[0, :N]

    order = jnp.argsort(-ssel)
    padi = jnp.full((NP - N,), 0, jnp.int32)
    order_p = jnp.concatenate([order.astype(jnp.int32), padi])
    valid = (jax.lax.iota(jnp.int32, NP) < N)

    x1f = jnp.where(valid, boxes[order_p, 0], 0.0)
    y1f = jnp.where(valid, boxes[order_p, 1], 0.0)
    x2f = jnp.where(valid, boxes[order_p, 2], 0.0)
    y2f = jnp.where(valid, boxes[order_p, 3], 0.0)
    clsf = jnp.where(valid, cls_inds[order_p], -1)

    def rc(v):
        return v.reshape(R, C)

    def col(v):
        return v.reshape(NP, 1)

    keep_s = pl.pallas_call(
        _nms_kernel,
        out_shape=jax.ShapeDtypeStruct((R, C), jnp.float32),
        scratch_shapes=[pltpu.VMEM((R, C), jnp.float32)],
    )(rc(x1f), rc(y1f), rc(x2f), rc(y2f), rc(clsf),
      col(x1f), col(y1f), col(x2f), col(y2f), col(clsf))

    keep_sorted = keep_s.reshape(NP)[:N]
    keep = jnp.zeros((N,), jnp.float32).at[order].set(keep_sorted)

    boxes_out = boxes * keep[:, None]
    scores_out = ssel * keep
    return boxes_out, scores_out, cls_inds


# speedup vs baseline: 737.7620x; 8.1763x over previous
import functools

import jax
import jax.numpy as jnp
from jax.experimental import pallas as pl
from jax.experimental.pallas import tpu as pltpu
from jax.experimental.pallas import tpu_sc as plsc

N = 5000
NUM_CLASSES = 20
NP = 5120
L = 16
NTILES = 32
NCHUNKS = NP // L
THRESH = 0.5


def _cls_kernel(st_ref, cls_ref, ssel_ref):
    s = st_ref[...]
    m = jnp.max(s, axis=0, keepdims=True)
    row = jax.lax.broadcasted_iota(jnp.int32, s.shape, 0)
    idx = jnp.min(jnp.where(s == m, row, NUM_CLASSES), axis=0, keepdims=True)
    cls_ref[...] = idx
    ssel_ref[...] = m


def _sc_nms(x1h, y1h, x2h, y2h, ordh, sth, cnth, out_ref,
            x1v, y1v, x2v, y2v, ordv, stv, cntv,
            lx1, ly1, lx2, ly2, lar, lidx, suppv, keeprow):
    wid = jax.lax.axis_index("s") * 2 + jax.lax.axis_index("c")
    iota = jax.lax.iota(jnp.int32, L)

    def zero_body(k, _):
        keeprow[pl.ds(k * L, L)] = jnp.zeros((L,), jnp.float32)
        return 0

    jax.lax.fori_loop(0, NCHUNKS, zero_body, 0)

    pltpu.sync_copy(x1h, x1v)
    pltpu.sync_copy(y1h, y1v)
    pltpu.sync_copy(x2h, x2v)
    pltpu.sync_copy(y2h, y2v)
    pltpu.sync_copy(ordh, ordv)
    pltpu.sync_copy(sth, stv)
    pltpu.sync_copy(cnth, cntv)

    def sload(ref, i):
        v = plsc.load_gather(ref, [jnp.full((L,), i, jnp.int32)])
        return v[0]

    start = sload(stv, wid)
    n = sload(cntv, wid)
    nch = (n + L - 1) // L

    def gather_body(k, _):
        p16 = jnp.full((L,), start + k * L, jnp.int32) + iota
        idx16 = plsc.load_gather(ordv, [p16])
        lidx[pl.ds(k * L, L)] = idx16
        a = plsc.load_gather(x1v, [idx16])
        b = plsc.load_gather(y1v, [idx16])
        c = plsc.load_gather(x2v, [idx16])
        d = plsc.load_gather(y2v, [idx16])
        lx1[pl.ds(k * L, L)] = a
        ly1[pl.ds(k * L, L)] = b
        lx2[pl.ds(k * L, L)] = c
        ly2[pl.ds(k * L, L)] = d
        lar[pl.ds(k * L, L)] = (c - a) * (d - b)
        suppv[pl.ds(k * L, L)] = jnp.zeros((L,), jnp.float32)
        return 0

    jax.lax.fori_loop(0, nch, gather_body, 0)

    def outer(i, _):
        ii = jnp.full((L,), i, jnp.int32)
        si = plsc.load_gather(suppv, [ii])
        act = si[0] == 0.0

        @pl.when(act)
        def _sweep():
            x1i = plsc.load_gather(lx1, [ii])
            y1i = plsc.load_gather(ly1, [ii])
            x2i = plsc.load_gather(lx2, [ii])
            y2i = plsc.load_gather(ly2, [ii])
            ai = plsc.load_gather(lar, [ii])

            def inner(k, _2):
                b0 = k * L
                xx1 = jnp.maximum(lx1[pl.ds(b0, L)], x1i)
                yy1 = jnp.maximum(ly1[pl.ds(b0, L)], y1i)
                xx2 = jnp.minimum(lx2[pl.ds(b0, L)], x2i)
                yy2 = jnp.minimum(ly2[pl.ds(b0, L)], y2i)
                w = jnp.maximum(1e-10, xx2 - xx1)
                h = jnp.maximum(1e-10, yy2 - yy1)
                inter = w * h
                over = inter / ((ai + lar[pl.ds(b0, L)]) - inter)
                pos = jnp.full((L,), b0, jnp.int32) + iota
                ns = (over > THRESH) & (pos > i)
                suppv[pl.ds(b0, L)] = jnp.maximum(
                    suppv[pl.ds(b0, L)], ns.astype(jnp.float32))
                return 0

            jax.lax.fori_loop(i // L, nch, inner, 0)

        return 0

    jax.lax.fori_loop(0, n, outer, 0)

    def scatter_body(k, _):
        idx16 = lidx[pl.ds(k * L, L)]
        sp = suppv[pl.ds(k * L, L)]
        pos = jnp.full((L,), k * L, jnp.int32) + iota
        m = pos < n
        plsc.store_scatter(keeprow, [idx16], 1.0 - sp, mask=m)
        return 0

    jax.lax.fori_loop(0, nch, scatter_body, 0)

    pltpu.sync_copy(keeprow, out_ref.at[wid])


def _combine_kernel(rows_ref, bt_ref, ssel_ref, bo_ref, so_ref):
    keep = jnp.max(rows_ref[...], axis=0, keepdims=True)
    bo_ref[...] = bt_ref[...] * keep
    so_ref[...] = ssel_ref[...] * keep


@jax.jit
def kernel(boxes, scores):
    st = jnp.zeros((NUM_CLASSES, NP), jnp.float32)
    st = st.at[:, :N].set(scores.T)
    cls_p, ssel_p = pl.pallas_call(
        _cls_kernel,
        out_shape=[
            jax.ShapeDtypeStruct((1, NP), jnp.int32),
            jax.ShapeDtypeStruct((1, NP), jnp.float32),
        ],
    )(st)
    cls_inds = cls_p[0, :N]
    ssel = ssel_p[0, :N]

    key = cls_inds.astype(jnp.float32) * 2.0 - ssel
    order = jnp.argsort(key).astype(jnp.int32)
    order_p = jnp.concatenate([order, jnp.zeros((NP - N,), jnp.int32)])
    counts = jnp.sum(
        (cls_inds[None, :] == jnp.arange(NUM_CLASSES, dtype=jnp.int32)[:, None])
        .astype(jnp.int32), axis=1)
    starts = jnp.concatenate(
        [jnp.zeros((1,), jnp.int32), jnp.cumsum(counts)[:-1].astype(jnp.int32)])
    starts32 = jnp.concatenate(
        [starts, jnp.full((128 - NUM_CLASSES,), N, jnp.int32)])
    counts32 = jnp.concatenate(
        [counts, jnp.zeros((128 - NUM_CLASSES,), jnp.int32)])

    def padnp(v):
        return jnp.concatenate([v, jnp.zeros((NP - N,), v.dtype)])

    x1p = padnp(boxes[:, 0])
    y1p = padnp(boxes[:, 1])
    x2p = padnp(boxes[:, 2])
    y2p = padnp(boxes[:, 3])

    mesh = plsc.VectorSubcoreMesh(core_axis_name="c", subcore_axis_name="s")
    keep_rows = pl.kernel(
        _sc_nms,
        out_type=jax.ShapeDtypeStruct((NTILES, NP), jnp.float32),
        mesh=mesh,
        compiler_params=pltpu.CompilerParams(needs_layout_passes=False),
        scratch_types=[
            pltpu.VMEM((NP,), jnp.float32),
            pltpu.VMEM((NP,), jnp.float32),
            pltpu.VMEM((NP,), jnp.float32),
            pltpu.VMEM((NP,), jnp.float32),
            pltpu.VMEM((NP,), jnp.int32),
            pltpu.VMEM((128,), jnp.int32),
            pltpu.VMEM((128,), jnp.int32),
            pltpu.VMEM((NP,), jnp.float32),
            pltpu.VMEM((NP,), jnp.float32),
            pltpu.VMEM((NP,), jnp.float32),
            pltpu.VMEM((NP,), jnp.float32),
            pltpu.VMEM((NP,), jnp.float32),
            pltpu.VMEM((NP,), jnp.int32),
            pltpu.VMEM((NP,), jnp.float32),
            pltpu.VMEM((NP,), jnp.float32),
        ],
    )(x1p, y1p, x2p, y2p, order_p, starts32, counts32)

    bt = jnp.zeros((4, NP), jnp.float32)
    bt = bt.at[:, :N].set(boxes.T)
    bo, so = pl.pallas_call(
        _combine_kernel,
        out_shape=[
            jax.ShapeDtypeStruct((4, NP), jnp.float32),
            jax.ShapeDtypeStruct((1, NP), jnp.float32),
        ],
    )(keep_rows, bt, ssel_p)

    boxes_out = bo[:, :N].T
    scores_out = so[0, :N]
    return boxes_out, scores_out, cls_inds


# speedup vs baseline: 1025.5765x; 1.3901x over previous
import functools

import jax
import jax.numpy as jnp
from jax.experimental import pallas as pl
from jax.experimental.pallas import tpu as pltpu
from jax.experimental.pallas import tpu_sc as plsc

N = 5000
NUM_CLASSES = 20
NP = 5120
L = 16
NTILES = 32
NCHUNKS = NP // L
THRESH = 0.5


def _cls_kernel(st_ref, cls_ref, ssel_ref):
    s = st_ref[...]
    m = jnp.max(s, axis=0, keepdims=True)
    row = jax.lax.broadcasted_iota(jnp.int32, s.shape, 0)
    idx = jnp.min(jnp.where(s == m, row, NUM_CLASSES), axis=0, keepdims=True)
    cls_ref[...] = idx
    ssel_ref[...] = m


def _sc_nms(x1h, y1h, x2h, y2h, ordh, sth, cnth, out_ref,
            x1v, y1v, x2v, y2v, ordv, stv, cntv,
            lx1, ly1, lx2, ly2, lar, lidx, suppv, keeprow):
    wid = jax.lax.axis_index("s") * 2 + jax.lax.axis_index("c")
    iota = jax.lax.iota(jnp.int32, L)

    def zero_body(k, _):
        keeprow[pl.ds(k * L, L)] = jnp.zeros((L,), jnp.float32)
        return 0

    jax.lax.fori_loop(0, NCHUNKS, zero_body, 0)

    pltpu.sync_copy(x1h, x1v)
    pltpu.sync_copy(y1h, y1v)
    pltpu.sync_copy(x2h, x2v)
    pltpu.sync_copy(y2h, y2v)
    pltpu.sync_copy(ordh, ordv)
    pltpu.sync_copy(sth, stv)
    pltpu.sync_copy(cnth, cntv)

    def sload(ref, i):
        v = plsc.load_gather(ref, [jnp.full((L,), i, jnp.int32)])
        return v[0]

    start = sload(stv, wid)
    n = sload(cntv, wid)
    nch = (n + L - 1) // L

    def gather_body(k, _):
        p16 = jnp.full((L,), start + k * L, jnp.int32) + iota
        idx16 = plsc.load_gather(ordv, [p16])
        lidx[pl.ds(k * L, L)] = idx16
        a = plsc.load_gather(x1v, [idx16])
        b = plsc.load_gather(y1v, [idx16])
        c = plsc.load_gather(x2v, [idx16])
        d = plsc.load_gather(y2v, [idx16])
        lx1[pl.ds(k * L, L)] = a
        ly1[pl.ds(k * L, L)] = b
        lx2[pl.ds(k * L, L)] = c
        ly2[pl.ds(k * L, L)] = d
        lar[pl.ds(k * L, L)] = (c - a) * (d - b)
        suppv[pl.ds(k * L, L)] = jnp.zeros((L,), jnp.float32)
        return 0

    jax.lax.fori_loop(0, nch, gather_body, 0)

    def outer(i, _):
        ii = jnp.full((L,), i, jnp.int32)
        si = plsc.load_gather(suppv, [ii])
        act = si[0] == 0.0

        @pl.when(act)
        def _sweep():
            x1i = plsc.load_gather(lx1, [ii])
            y1i = plsc.load_gather(ly1, [ii])
            x2i = plsc.load_gather(lx2, [ii])
            y2i = plsc.load_gather(ly2, [ii])
            ai = plsc.load_gather(lar, [ii])

            def overlap(b0):
                xx1 = jnp.maximum(lx1[pl.ds(b0, L)], x1i)
                yy1 = jnp.maximum(ly1[pl.ds(b0, L)], y1i)
                xx2 = jnp.minimum(lx2[pl.ds(b0, L)], x2i)
                yy2 = jnp.minimum(ly2[pl.ds(b0, L)], y2i)
                w = jnp.maximum(1e-10, xx2 - xx1)
                h = jnp.maximum(1e-10, yy2 - yy1)
                inter = w * h
                denom = (ai + lar[pl.ds(b0, L)]) - inter
                return (inter > THRESH * denom) & (denom >= 0.0)

            k0 = i // L
            b0 = k0 * L
            pos = jnp.full((L,), b0, jnp.int32) + iota
            ns0 = overlap(b0) & (pos > i)
            suppv[pl.ds(b0, L)] = jnp.maximum(
                suppv[pl.ds(b0, L)], ns0.astype(jnp.float32))

            @plsc.parallel_loop(k0 + 1, nch, unroll=2)
            def _rest(k):
                b = k * L
                ns = overlap(b)
                suppv[pl.ds(b, L)] = jnp.maximum(
                    suppv[pl.ds(b, L)], ns.astype(jnp.float32))

        return 0

    jax.lax.fori_loop(0, n, outer, 0)

    def scatter_body(k, _):
        idx16 = lidx[pl.ds(k * L, L)]
        sp = suppv[pl.ds(k * L, L)]
        pos = jnp.full((L,), k * L, jnp.int32) + iota
        m = pos < n
        plsc.store_scatter(keeprow, [idx16], 1.0 - sp, mask=m)
        return 0

    jax.lax.fori_loop(0, nch, scatter_body, 0)

    pltpu.sync_copy(keeprow, out_ref.at[wid])


def _combine_kernel(rows_ref, bt_ref, ssel_ref, bo_ref, so_ref):
    keep = jnp.max(rows_ref[...], axis=0, keepdims=True)
    bo_ref[...] = bt_ref[...] * keep
    so_ref[...] = ssel_ref[...] * keep


@jax.jit
def kernel(boxes, scores):
    st = jnp.zeros((NUM_CLASSES, NP), jnp.float32)
    st = st.at[:, :N].set(scores.T)
    cls_p, ssel_p = pl.pallas_call(
        _cls_kernel,
        out_shape=[
            jax.ShapeDtypeStruct((1, NP), jnp.int32),
            jax.ShapeDtypeStruct((1, NP), jnp.float32),
        ],
    )(st)
    cls_inds = cls_p[0, :N]
    ssel = ssel_p[0, :N]

    key = cls_inds.astype(jnp.float32) * 2.0 - ssel
    order = jnp.argsort(key).astype(jnp.int32)
    order_p = jnp.concatenate([order, jnp.zeros((NP - N,), jnp.int32)])
    counts = jnp.sum(
        (cls_inds[None, :] == jnp.arange(NUM_CLASSES, dtype=jnp.int32)[:, None])
        .astype(jnp.int32), axis=1)
    starts = jnp.concatenate(
        [jnp.zeros((1,), jnp.int32), jnp.cumsum(counts)[:-1].astype(jnp.int32)])
    starts32 = jnp.concatenate(
        [starts, jnp.full((128 - NUM_CLASSES,), N, jnp.int32)])
    counts32 = jnp.concatenate(
        [counts, jnp.zeros((128 - NUM_CLASSES,), jnp.int32)])

    def padnp(v):
        return jnp.concatenate([v, jnp.zeros((NP - N,), v.dtype)])

    x1p = padnp(boxes[:, 0])
    y1p = padnp(boxes[:, 1])
    x2p = padnp(boxes[:, 2])
    y2p = padnp(boxes[:, 3])

    mesh = plsc.VectorSubcoreMesh(core_axis_name="c", subcore_axis_name="s")
    keep_rows = pl.kernel(
        _sc_nms,
        out_type=jax.ShapeDtypeStruct((NTILES, NP), jnp.float32),
        mesh=mesh,
        compiler_params=pltpu.CompilerParams(needs_layout_passes=False),
        scratch_types=[
            pltpu.VMEM((NP,), jnp.float32),
            pltpu.VMEM((NP,), jnp.float32),
            pltpu.VMEM((NP,), jnp.float32),
            pltpu.VMEM((NP,), jnp.float32),
            pltpu.VMEM((NP,), jnp.int32),
            pltpu.VMEM((128,), jnp.int32),
            pltpu.VMEM((128,), jnp.int32),
            pltpu.VMEM((NP,), jnp.float32),
            pltpu.VMEM((NP,), jnp.float32),
            pltpu.VMEM((NP,), jnp.float32),
            pltpu.VMEM((NP,), jnp.float32),
            pltpu.VMEM((NP,), jnp.float32),
            pltpu.VMEM((NP,), jnp.int32),
            pltpu.VMEM((NP,), jnp.float32),
            pltpu.VMEM((NP,), jnp.float32),
        ],
    )(x1p, y1p, x2p, y2p, order_p, starts32, counts32)

    bt = jnp.zeros((4, NP), jnp.float32)
    bt = bt.at[:, :N].set(boxes.T)
    bo, so = pl.pallas_call(
        _combine_kernel,
        out_shape=[
            jax.ShapeDtypeStruct((4, NP), jnp.float32),
            jax.ShapeDtypeStruct((1, NP), jnp.float32),
        ],
    )(keep_rows, bt, ssel_p)

    boxes_out = bo[:, :N].T
    scores_out = so[0, :N]
    return boxes_out, scores_out, cls_inds
